# Initial kernel scaffold; baseline (speedup 1.0000x reference)
import jax
import jax.numpy as jnp
from jax.experimental import pallas as pl


def kernel(x, edge_index, edge_attr, W1, b1, W2, b2):
    raise NotImplementedError("write your pallas kernel here")


# speedup vs baseline: 2.7343x; 2.7343x over previous
import functools

import jax
import jax.numpy as jnp
from jax import lax
from jax.experimental import pallas as pl
from jax.experimental.pallas import tpu as pltpu
from jax.experimental.pallas import tpu_sc as plsc

N_NODES = 10000
N_EDGES = 320000
D_FEAT = 128
D_EDGE = 16
D_HIDDEN = 512

NC = 2
NS = 16
NW = NC * NS
EPW = N_EDGES // NW
CH = 80
NCH = EPW // CH
N_PAD = 10240
NPT = N_PAD // NS

_sc_mesh = plsc.VectorSubcoreMesh(core_axis_name="c", subcore_axis_name="s")


def _gather_body(x_hbm, dst_hbm, src_hbm, gobj_hbm, gsub_hbm,
                 idx_d, idx_s, buf_d, buf_s, sem_d, sem_s):
    cid = lax.axis_index("c")
    sid = lax.axis_index("s")
    wid = sid * NC + cid
    pltpu.sync_copy(dst_hbm.at[wid], idx_d)
    pltpu.sync_copy(src_hbm.at[wid], idx_s)

    def body(j, carry):
        cp_d = pltpu.async_copy(x_hbm.at[idx_d.at[j]], buf_d, sem_d)
        cp_s = pltpu.async_copy(x_hbm.at[idx_s.at[j]], buf_s, sem_s)
        cp_d.wait()
        pltpu.sync_copy(buf_d, gobj_hbm.at[wid, j])
        cp_s.wait()
        pltpu.sync_copy(buf_s, gsub_hbm.at[wid, j])
        return carry

    lax.fori_loop(0, NCH, body, 0)


@jax.jit
def _sc_gather(x, dst3, src3):
    out_t = jax.ShapeDtypeStruct((NW, NCH, CH, D_FEAT), jnp.float32)
    return pl.kernel(
        _gather_body,
        out_type=(out_t, out_t),
        mesh=_sc_mesh,
        scratch_types=[
            pltpu.VMEM((NCH, CH), jnp.int32),
            pltpu.VMEM((NCH, CH), jnp.int32),
            pltpu.VMEM((CH, D_FEAT), jnp.float32),
            pltpu.VMEM((CH, D_FEAT), jnp.float32),
            pltpu.SemaphoreType.DMA,
            pltpu.SemaphoreType.DMA,
        ],
    )(x, dst3, src3)


def _scatter_body(msg_hbm, dst_hbm, zero_hbm, out_hbm,
                  idx_d, buf, acc, sem):
    cid = lax.axis_index("c")
    sid = lax.axis_index("s")
    wid = sid * NC + cid
    pltpu.sync_copy(zero_hbm.at[pl.ds(sid * NPT, NPT)],
                    acc.at[pl.ds(sid * NPT, NPT)])
    pltpu.sync_copy(dst_hbm.at[wid], idx_d)
    plsc.subcore_barrier()

    def body(j, carry):
        pltpu.sync_copy(msg_hbm.at[wid, j], buf)
        pltpu.sync_copy(buf, acc.at[idx_d.at[j]], add=True)
        return carry

    lax.fori_loop(0, NCH, body, 0)
    plsc.subcore_barrier()
    pltpu.sync_copy(acc.at[pl.ds(sid * NPT, NPT)],
                    out_hbm.at[cid, pl.ds(sid * NPT, NPT)])


@jax.jit
def _sc_scatter(msg4, dst3, zeros_nm):
    return pl.kernel(
        _scatter_body,
        out_type=jax.ShapeDtypeStruct((NC, N_PAD, D_FEAT), jnp.float32),
        mesh=_sc_mesh,
        scratch_types=[
            pltpu.VMEM((NCH, CH), jnp.int32),
            pltpu.VMEM((CH, D_FEAT), jnp.float32),
            pltpu.VMEM_SHARED((N_PAD, D_FEAT), jnp.float32),
            pltpu.SemaphoreType.DMA,
        ],
    )(msg4, dst3, zeros_nm)


def _count_body(dst_hbm, zero_hbm, ones_hbm, out_hbm, idx_d, buf, acc, sem):
    cid = lax.axis_index("c")
    sid = lax.axis_index("s")
    wid = sid * NC + cid
    pltpu.sync_copy(zero_hbm.at[pl.ds(sid * NPT, NPT)],
                    acc.at[pl.ds(sid * NPT, NPT)])
    pltpu.sync_copy(dst_hbm.at[wid], idx_d)
    pltpu.sync_copy(ones_hbm, buf)
    plsc.subcore_barrier()

    def body(j, carry):
        pltpu.sync_copy(buf, acc.at[idx_d.at[j]], add=True)
        return carry

    lax.fori_loop(0, NCH, body, 0)
    plsc.subcore_barrier()
    pltpu.sync_copy(acc.at[pl.ds(sid * NPT, NPT)],
                    out_hbm.at[cid, pl.ds(sid * NPT, NPT)])


@jax.jit
def _sc_count(dst3, zeros_nm, ones_ch):
    return pl.kernel(
        _count_body,
        out_type=jax.ShapeDtypeStruct((NC, N_PAD, D_FEAT), jnp.float32),
        mesh=_sc_mesh,
        scratch_types=[
            pltpu.VMEM((NCH, CH), jnp.int32),
            pltpu.VMEM((CH, D_FEAT), jnp.float32),
            pltpu.VMEM_SHARED((N_PAD, D_FEAT), jnp.float32),
            pltpu.SemaphoreType.DMA,
        ],
    )(dst3, zeros_nm, ones_ch)


BE = 512
assert N_EDGES % BE == 0


def _mlp_body(gobj, gsub, ea, w1a, w1e, w1b, b1, w2, b2, out):
    h = jnp.dot(gobj[...], w1a[...], preferred_element_type=jnp.float32)
    h = h + jnp.dot(ea[...], w1e[...], preferred_element_type=jnp.float32)
    h = h + jnp.dot(gsub[...], w1b[...], preferred_element_type=jnp.float32)
    h = jnp.maximum(h + b1[...], 0.0)
    out[...] = jnp.dot(h, w2[...], preferred_element_type=jnp.float32) + b2[...]


@jax.jit
def _tc_mlp(gobj, gsub, ea, w1a, w1e, w1b, b1, w2, b2):
    grid = (N_EDGES // BE,)
    full = lambda shape: pl.BlockSpec(shape, lambda i: (0, 0))
    return pl.pallas_call(
        _mlp_body,
        grid=grid,
        in_specs=[
            pl.BlockSpec((BE, D_FEAT), lambda i: (i, 0)),
            pl.BlockSpec((BE, D_FEAT), lambda i: (i, 0)),
            pl.BlockSpec((BE, D_EDGE), lambda i: (i, 0)),
            full((D_FEAT, D_HIDDEN)),
            full((D_EDGE, D_HIDDEN)),
            full((D_FEAT, D_HIDDEN)),
            full((1, D_HIDDEN)),
            full((D_HIDDEN, D_FEAT)),
            full((1, D_FEAT)),
        ],
        out_specs=pl.BlockSpec((BE, D_FEAT), lambda i: (i, 0)),
        out_shape=jax.ShapeDtypeStruct((N_EDGES, D_FEAT), jnp.float32),
    )(gobj, gsub, ea, w1a, w1e, w1b, b1, w2, b2)


def _combine_body(s_ref, c_ref, out_ref):
    s = s_ref[0, :N_NODES, :] + s_ref[1, :N_NODES, :]
    c = jnp.maximum(c_ref[0, :N_NODES, :] + c_ref[1, :N_NODES, :], 1.0)
    out_ref[...] = s / c


@jax.jit
def _tc_combine(partials, counts):
    return pl.pallas_call(
        _combine_body,
        out_shape=jax.ShapeDtypeStruct((N_NODES, D_FEAT), jnp.float32),
    )(partials, counts)


def kernel(x, edge_index, edge_attr, W1, b1, W2, b2):
    dst = edge_index[1].astype(jnp.int32)
    src = edge_index[0].astype(jnp.int32)
    dst3 = dst.reshape(NW, NCH, CH)
    src3 = src.reshape(NW, NCH, CH)

    gobj4, gsub4 = _sc_gather(x, dst3, src3)
    gobj = gobj4.reshape(N_EDGES, D_FEAT)
    gsub = gsub4.reshape(N_EDGES, D_FEAT)

    w1a = W1[:D_FEAT]
    w1e = W1[D_FEAT:D_FEAT + D_EDGE]
    w1b = W1[D_FEAT + D_EDGE:]
    msg = _tc_mlp(gobj, gsub, edge_attr,
                  w1a, w1e, w1b, b1.reshape(1, D_HIDDEN),
                  W2, b2.reshape(1, D_FEAT))

    zeros_nm = jnp.zeros((N_PAD, D_FEAT), jnp.float32)
    ones_ch = jnp.ones((CH, D_FEAT), jnp.float32)
    partials = _sc_scatter(msg.reshape(NW, NCH, CH, D_FEAT), dst3, zeros_nm)
    counts = _sc_count(dst3, zeros_nm, ones_ch)
    return _tc_combine(partials, counts)
